# Initial kernel scaffold; baseline (speedup 1.0000x reference)
import jax
import jax.numpy as jnp
from jax.experimental import pallas as pl


def kernel(eta, alpha, phi, words, doc_ids, z_old):
    raise NotImplementedError("write your pallas kernel here")


# speedup vs baseline: 12.0716x; 12.0716x over previous
import functools

import jax
import jax.numpy as jnp
from jax import lax
from jax.experimental import pallas as pl
from jax.experimental.pallas import tpu as pltpu
from jax.experimental.pallas import tpu_sc as plsc

K = 64
V = 100000
D = 4096
NTOK = 524288
ETA_VAR = 0.05
SGLD_A = 0.01
SGLD_B = 1.0
SGLD_C = 0.55
ZERO = 1e-06

NC = 2
NS = 16
NW = NC * NS
L = 16

TOK_W = NTOK // NW
DK = D * K
SH_W = DK // NS

_mesh = plsc.VectorSubcoreMesh(core_axis_name="c", subcore_axis_name="s")


def _jexp(x):
    return jnp.maximum(jnp.exp(jnp.clip(x, -700.0, 700.0)), ZERO)


@functools.partial(
    pl.kernel,
    out_type=jax.ShapeDtypeStruct((NC, DK), jnp.float32),
    mesh=_mesh,
    scratch_types=[
        pltpu.VMEM((TOK_W,), jnp.int32),
        pltpu.VMEM((TOK_W,), jnp.int32),
        pltpu.VMEM((TOK_W,), jnp.int32),
        pltpu.VMEM((TOK_W,), jnp.float32),
        pltpu.VMEM_SHARED((DK,), jnp.float32),
    ],
)
def _sc_cdk(doc_hbm, z_hbm, out_hbm, doc_v, z_v, idx_v, val_v, shared):
    cid = lax.axis_index("c")
    sid = lax.axis_index("s")
    wid = sid * NC + cid
    base = wid * TOK_W

    def _zero(i, _):
        val_v[pl.ds(i * L, L)] = jnp.zeros((L,), jnp.float32)
        return 0
    lax.fori_loop(0, TOK_W // L, _zero, 0)
    pltpu.sync_copy(val_v, shared.at[pl.ds(sid * SH_W, SH_W)])
    plsc.subcore_barrier()

    pltpu.sync_copy(doc_hbm.at[pl.ds(base, TOK_W)], doc_v)
    pltpu.sync_copy(z_hbm.at[pl.ds(base, TOK_W)], z_v)

    def _mkidx(i, _):
        s = pl.ds(i * L, L)
        idx_v[s] = doc_v[s] * K + z_v[s]
        val_v[s] = jnp.full((L,), 1.0, jnp.float32)
        return 0
    lax.fori_loop(0, TOK_W // L, _mkidx, 0)

    pltpu.sync_copy(val_v, shared.at[idx_v], add=True)
    plsc.subcore_barrier()

    sh = pl.ds(sid * SH_W, SH_W)
    pltpu.sync_copy(shared.at[sh], val_v)
    pltpu.sync_copy(val_v, out_hbm.at[cid].at[sh])


_EPS = SGLD_A * (SGLD_B + 1.0) ** (-SGLD_C)
_BD = 512


def _eta_body(eta_ref, alpha_ref, cdk_ref, xi_ref, out_ref):
    eta = eta_ref[...]
    cdk = cdk_ref[0] + cdk_ref[1]
    nd = jnp.sum(cdk, axis=1, keepdims=True)
    m = jnp.max(eta, axis=1, keepdims=True)
    ex = jnp.exp(eta - m)
    sm = ex / jnp.sum(ex, axis=1, keepdims=True)
    prior = (alpha_ref[...] - eta) / ETA_VAR
    grad = cdk - nd * sm
    out_ref[...] = eta + (_EPS / 2.0) * (grad + prior) + xi_ref[...]


def _tc_eta(eta, alpha, cdk2, xi):
    return pl.pallas_call(
        _eta_body,
        grid=(D // _BD,),
        in_specs=[
            pl.BlockSpec((_BD, K), lambda i: (i, 0)),
            pl.BlockSpec((1, K), lambda i: (0, 0)),
            pl.BlockSpec((NC, _BD, K), lambda i: (0, i, 0)),
            pl.BlockSpec((_BD, K), lambda i: (i, 0)),
        ],
        out_specs=pl.BlockSpec((_BD, K), lambda i: (i, 0)),
        out_shape=jax.ShapeDtypeStruct((D, K), jnp.float32),
    )(eta, alpha.reshape(1, K), cdk2, xi)


C = 4096
NCH = TOK_W // C
C3 = 3 * C


@functools.partial(
    pl.kernel,
    out_type=jax.ShapeDtypeStruct((NTOK,), jnp.float32),
    mesh=_mesh,
    scratch_types=[
        pltpu.VMEM((C,), jnp.int32),
        pltpu.VMEM((C,), jnp.int32),
        pltpu.VMEM((C,), jnp.int32),
        pltpu.VMEM((C,), jnp.int32),
        pltpu.VMEM((C,), jnp.int32),
        pltpu.VMEM((C,), jnp.float32),
        pltpu.VMEM((C,), jnp.float32),
        pltpu.VMEM((C3,), jnp.int32),
        pltpu.VMEM((C3,), jnp.int32),
        pltpu.VMEM((C3,), jnp.float32),
        pltpu.VMEM((C3,), jnp.float32),
        pltpu.VMEM((C,), jnp.float32),
        pltpu.SemaphoreType.DMA,
    ],
)
def _sc_tokens(phi_hbm, eta_hbm, w_hbm, d_hbm, zo_hbm, p1_hbm, p2_hbm,
               u1_hbm, u2_hbm, out_hbm,
               w_v, d_v, zo_v, p1_v, p2_v, u1_v, u2_v,
               pidx_v, eidx_v, pg_v, eg_v, o_v, sem):
    cid = lax.axis_index("c")
    sid = lax.axis_index("s")
    wid = sid * NC + cid
    base = wid * TOK_W

    def _chunk(ch, _):
        off = base + ch * C
        s_in = pl.ds(off, C)
        pltpu.sync_copy(w_hbm.at[s_in], w_v)
        pltpu.sync_copy(d_hbm.at[s_in], d_v)
        pltpu.sync_copy(zo_hbm.at[s_in], zo_v)
        pltpu.sync_copy(p1_hbm.at[s_in], p1_v)
        pltpu.sync_copy(p2_hbm.at[s_in], p2_v)
        pltpu.sync_copy(u1_hbm.at[s_in], u1_v)
        pltpu.sync_copy(u2_hbm.at[s_in], u2_v)

        def _mkidx(i, _):
            s = pl.ds(i * L, L)
            s1 = pl.ds(C + i * L, L)
            s2 = pl.ds(2 * C + i * L, L)
            wK = w_v[s] * K
            dK = d_v[s] * K
            pidx_v[s] = wK + zo_v[s]
            pidx_v[s1] = wK + p1_v[s]
            pidx_v[s2] = wK + p2_v[s]
            eidx_v[s] = dK + zo_v[s]
            eidx_v[s1] = dK + p1_v[s]
            eidx_v[s2] = dK + p2_v[s]
            return 0
        lax.fori_loop(0, C // L, _mkidx, 0)

        cp1 = pltpu.async_copy(phi_hbm.at[pidx_v], pg_v, sem)
        cp2 = pltpu.async_copy(eta_hbm.at[eidx_v], eg_v, sem)
        cp1.wait()
        cp2.wait()

        def _mh(i, _):
            s = pl.ds(i * L, L)
            s1 = pl.ds(C + i * L, L)
            s2 = pl.ds(2 * C + i * L, L)
            a = pg_v[s]
            b = pg_v[s1]
            e = pg_v[s2]
            f = eg_v[s]
            g = eg_v[s1]
            c = eg_v[s2]
            acc1 = _jexp(b) / _jexp(a)
            rej1 = u1_v[s] >= acc1
            dval = jnp.where(rej1, f, g)
            acc2 = _jexp(c) / _jexp(dval)
            rej2 = u2_v[s] >= acc2
            phi_z2 = jnp.where(rej2, jnp.where(rej1, a, b), e)
            eta_z2 = jnp.where(rej2, dval, c)
            o_v[s] = eta_z2 + phi_z2
            return 0
        lax.fori_loop(0, C // L, _mh, 0)

        pltpu.sync_copy(o_v, out_hbm.at[s_in])
        return 0
    lax.fori_loop(0, NCH, _chunk, 0)


def kernel(eta, alpha, phi, words, doc_ids, z_old):
    key = jax.random.key(42)
    kxi, kp1, ku1, kp2, ku2 = jax.random.split(key, 5)
    eps = SGLD_A * (SGLD_B + 1.0) ** (-SGLD_C)
    xi = jax.random.normal(kxi, (D, K), dtype=jnp.float32) * eps
    prop1 = jax.random.randint(kp1, (NTOK,), 0, K - 1).astype(jnp.int32)
    u1 = jax.random.uniform(ku1, (NTOK,))
    prop2 = jax.random.randint(kp2, (NTOK,), 0, K - 1).astype(jnp.int32)
    u2 = jax.random.uniform(ku2, (NTOK,))

    words = words.astype(jnp.int32)
    doc_ids = doc_ids.astype(jnp.int32)
    z_old = z_old.astype(jnp.int32)

    cdk2 = _sc_cdk(doc_ids, z_old)
    eta_new = _tc_eta(eta, alpha, cdk2.reshape(NC, D, K), xi)
    logp = _sc_tokens(phi.reshape(-1), eta_new.reshape(-1),
                      words, doc_ids, z_old, prop1, prop2, u1, u2)
    return logp
